# baseline (speedup 1.0000x reference)
import jax
import jax.numpy as jnp
from jax.experimental import pallas as pl


def _touch_kernel(x_ref, o_ref):
    o_ref[...] = x_ref[...]


def _gcn_conv(x, edge_index, W, b):
    n = x.shape[0]
    self_idx = jnp.arange(n, dtype=edge_index.dtype)
    src = jnp.concatenate([edge_index[0], self_idx])
    dst = jnp.concatenate([edge_index[1], self_idx])
    deg = jax.ops.segment_sum(jnp.ones(src.shape[0], dtype=x.dtype), dst, num_segments=n)
    dinv = jnp.where(deg > 0, 1.0 / jnp.sqrt(deg), 0.0)
    norm = dinv[src] * dinv[dst]
    h = x @ W
    msg = jnp.take(h, src, axis=0) * norm[:, None]
    out = jax.ops.segment_sum(msg, dst, num_segments=n)
    return out + b


def _bn(x, gamma, beta, eps=1e-5):
    mu = jnp.mean(x, axis=0, keepdims=True)
    var = jnp.var(x, axis=0, keepdims=True)
    return (x - mu) / jnp.sqrt(var + eps) * gamma + beta


def kernel(x, fwd_edges_index, bwd_edges_index, edge_attr,
           fwd_W0, fwd_b0, bwd_W0, bwd_b0, mW0, mb0, g0, be0,
           fwd_W1, fwd_b1, bwd_W1, bwd_b1, mW1, mb1, g1, be1):
    layers = [(fwd_W0, fwd_b0, bwd_W0, bwd_b0, mW0, mb0, g0, be0),
              (fwd_W1, fwd_b1, bwd_W1, bwd_b1, mW1, mb1, g1, be1)]
    for (fW, fb, bW, bb, mW, mb, g, be) in layers:
        x0 = x
        fx = _gcn_conv(x, fwd_edges_index, fW, fb)
        bx = _gcn_conv(x, bwd_edges_index, bW, bb)
        x = jnp.concatenate([fx, bx], axis=1) @ mW + mb
        x = jax.nn.relu(x)
        x = _bn(x, g, be)
        x = x + x0
    x = pl.pallas_call(
        _touch_kernel,
        out_shape=jax.ShapeDtypeStruct(x.shape, x.dtype),
    )(x)
    return x


# speedup vs baseline: 15.1804x; 15.1804x over previous
import dataclasses
import functools

import jax
import jax.numpy as jnp
from jax import lax
from jax.experimental import pallas as pl
from jax.experimental.pallas import tpu as pltpu
from jax.experimental.pallas import tpu_sc as plsc

N = 10000
E = 320000
D = 128
NSC = 2
NSUB = 16
EPC = E // NSUB
CHUNK = 200
NCHUNK = EPC // CHUNK
NPAD = 10240
RPS = NPAD // NSUB

@functools.cache
def _get_mesh():
    return plsc.VectorSubcoreMesh(core_axis_name="c", subcore_axis_name="s",
                                  num_cores=NSC, num_subcores=NSUB)


def _sc_compiler_params():
    cp = pltpu.CompilerParams()
    if "needs_layout_passes" in pltpu.CompilerParams.__dataclass_fields__:
        cp = dataclasses.replace(cp, needs_layout_passes=False)
    return cp


def _hist_body(dst_hbm, zeros_hbm, ones_hbm, out_hbm, acc, dstv, ones, sem):
    cid = lax.axis_index("c")
    sid = lax.axis_index("s")

    pltpu.sync_copy(ones_hbm, ones)
    pltpu.sync_copy(zeros_hbm.at[pl.ds(sid * RPS, RPS)], acc.at[pl.ds(sid * RPS, RPS)])
    plsc.subcore_barrier()

    @pl.loop(0, NCHUNK)
    def _(j):
        base = cid * E + sid * EPC + j * CHUNK
        pltpu.sync_copy(dst_hbm.at[pl.ds(base, CHUNK)], dstv)
        pltpu.sync_copy(ones, acc.at[dstv], add=True)

    plsc.subcore_barrier()
    pltpu.sync_copy(acc.at[pl.ds(sid * RPS, RPS)],
                    out_hbm.at[pl.ds(cid * NPAD + sid * RPS, RPS)])


def _agg_body(tcat_hbm, src_hbm, dst_hbm, zeros_hbm, out_hbm, acc, srcv, dstv, rows, sem):
    cid = lax.axis_index("c")
    sid = lax.axis_index("s")

    pltpu.sync_copy(zeros_hbm.at[pl.ds(sid * RPS, RPS)], acc.at[pl.ds(sid * RPS, RPS)])
    plsc.subcore_barrier()

    @pl.loop(0, NCHUNK)
    def _(j):
        base = cid * E + sid * EPC + j * CHUNK
        pltpu.sync_copy(src_hbm.at[pl.ds(base, CHUNK)], srcv)
        pltpu.sync_copy(dst_hbm.at[pl.ds(base, CHUNK)], dstv)
        pltpu.async_copy(tcat_hbm.at[srcv], rows, sem).wait()
        pltpu.sync_copy(rows, acc.at[dstv], add=True)

    plsc.subcore_barrier()
    pltpu.sync_copy(acc.at[pl.ds(sid * RPS, RPS)],
                    out_hbm.at[pl.ds(cid * NPAD + sid * RPS, RPS)])


@jax.jit
def _sc_hist(dst_all, zeros128, ones128):
    k = pl.kernel(
        _hist_body,
        out_type=jax.ShapeDtypeStruct((2 * NPAD, D), jnp.float32),
        mesh=_get_mesh(),
        scratch_types=[
            pltpu.VMEM_SHARED((NPAD, D), jnp.float32),
            pltpu.VMEM((CHUNK,), jnp.int32),
            pltpu.VMEM((CHUNK, D), jnp.float32),
            pltpu.SemaphoreType.DMA,
        ],
        compiler_params=_sc_compiler_params(),
    )
    return k(dst_all, zeros128, ones128)


@jax.jit
def _sc_agg(tcat, src_all, dst_all, zeros128):
    k = pl.kernel(
        _agg_body,
        out_type=jax.ShapeDtypeStruct((2 * NPAD, D), jnp.float32),
        mesh=_get_mesh(),
        scratch_types=[
            pltpu.VMEM_SHARED((NPAD, D), jnp.float32),
            pltpu.VMEM((CHUNK,), jnp.int32),
            pltpu.VMEM((CHUNK,), jnp.int32),
            pltpu.VMEM((CHUNK, D), jnp.float32),
            pltpu.SemaphoreType.DMA,
        ],
        compiler_params=_sc_compiler_params(),
    )
    return k(tcat, src_all, dst_all, zeros128)


_BLK = 1000
_NBLK = N // _BLK


def _prep_body(x_ref, hist_ref, out_ref):
    dinv = lax.rsqrt(hist_ref[:, 0:1] + 1.0)
    out_ref[...] = dinv * x_ref[...]


@jax.jit
def _tc_prep(x, hist_all):
    return pl.pallas_call(
        _prep_body,
        grid=(2 * _NBLK,),
        in_specs=[
            pl.BlockSpec((_BLK, D), lambda i: (lax.rem(i, _NBLK), 0)),
            pl.BlockSpec((_BLK, D), lambda i: (i, 0)),
        ],
        out_specs=pl.BlockSpec((_BLK, D), lambda i: (i, 0)),
        out_shape=jax.ShapeDtypeStruct((2 * N, D), jnp.float32),
    )(x, hist_all)


def _merge_body(aggf_ref, aggb_ref, tf_ref, tb_ref, hf_ref, hb_ref,
                fW_ref, fb_ref, bW_ref, bb_ref, mW_ref, mb_ref,
                m_ref, stats_ref, acc_ref):
    i = pl.program_id(0)
    dinvf = lax.rsqrt(hf_ref[:, 0:1] + 1.0)
    dinvb = lax.rsqrt(hb_ref[:, 0:1] + 1.0)
    convf = dinvf * jnp.dot(aggf_ref[...] + tf_ref[...], fW_ref[...],
                            preferred_element_type=jnp.float32) + fb_ref[...]
    convb = dinvb * jnp.dot(aggb_ref[...] + tb_ref[...], bW_ref[...],
                            preferred_element_type=jnp.float32) + bb_ref[...]
    m = (jnp.dot(convf, mW_ref[0:D, :], preferred_element_type=jnp.float32)
         + jnp.dot(convb, mW_ref[D:2 * D, :], preferred_element_type=jnp.float32)
         + mb_ref[...])
    m = jnp.maximum(m, 0.0)
    m_ref[...] = m

    @pl.when(i == 0)
    def _():
        acc_ref[...] = jnp.zeros_like(acc_ref)

    s = jnp.sum(m, axis=0, keepdims=True)
    s2 = jnp.sum(m * m, axis=0, keepdims=True)
    acc_ref[0:1, :] += s
    acc_ref[1:2, :] += s2
    stats_ref[...] = acc_ref[...]


@jax.jit
def _tc_merge(agg_all, tcat, hist_all, fW, fb, bW, bb, mW, mb):
    fb2 = fb.reshape(1, D)
    bb2 = bb.reshape(1, D)
    mb2 = mb.reshape(1, D)
    return pl.pallas_call(
        _merge_body,
        grid=(_NBLK,),
        in_specs=[
            pl.BlockSpec((_BLK, D), lambda i: (i, 0)),
            pl.BlockSpec((_BLK, D), lambda i: (i, 0)),
            pl.BlockSpec((_BLK, D), lambda i: (i, 0)),
            pl.BlockSpec((_BLK, D), lambda i: (i, 0)),
            pl.BlockSpec((_BLK, D), lambda i: (i, 0)),
            pl.BlockSpec((_BLK, D), lambda i: (i, 0)),
            pl.BlockSpec((D, D), lambda i: (0, 0)),
            pl.BlockSpec((1, D), lambda i: (0, 0)),
            pl.BlockSpec((D, D), lambda i: (0, 0)),
            pl.BlockSpec((1, D), lambda i: (0, 0)),
            pl.BlockSpec((2 * D, D), lambda i: (0, 0)),
            pl.BlockSpec((1, D), lambda i: (0, 0)),
        ],
        out_specs=[
            pl.BlockSpec((_BLK, D), lambda i: (i, 0)),
            pl.BlockSpec((8, D), lambda i: (0, 0)),
        ],
        out_shape=[
            jax.ShapeDtypeStruct((N, D), jnp.float32),
            jax.ShapeDtypeStruct((8, D), jnp.float32),
        ],
        scratch_shapes=[pltpu.VMEM((8, D), jnp.float32)],
    )(agg_all[0:N], agg_all[NPAD:NPAD + N], tcat[0:N], tcat[N:2 * N],
      hist_all[0:N], hist_all[N:2 * N], fW, fb2, bW, bb2, mW, mb2)


def _bnres_body(m_ref, stats_ref, xin_ref, hf_ref, hb_ref, g_ref, be_ref,
                xout_ref, tf_ref, tb_ref, *, emit_tables):
    mu = stats_ref[0:1, :] * (1.0 / N)
    var = stats_ref[1:2, :] * (1.0 / N) - mu * mu
    y = (m_ref[...] - mu) * lax.rsqrt(var + 1e-5) * g_ref[...] + be_ref[...]
    xout = y + xin_ref[...]
    xout_ref[...] = xout
    if emit_tables:
        tf_ref[...] = lax.rsqrt(hf_ref[:, 0:1] + 1.0) * xout
        tb_ref[...] = lax.rsqrt(hb_ref[:, 0:1] + 1.0) * xout


@functools.partial(jax.jit, static_argnames=("emit_tables",))
def _tc_bnres(m, stats, xin, hist_all, g, be, emit_tables):
    g2 = g.reshape(1, D)
    be2 = be.reshape(1, D)
    out_specs = [pl.BlockSpec((_BLK, D), lambda i: (i, 0))]
    out_shape = [jax.ShapeDtypeStruct((N, D), jnp.float32)]
    if emit_tables:
        out_specs += [pl.BlockSpec((_BLK, D), lambda i: (i, 0)),
                      pl.BlockSpec((_BLK, D), lambda i: (i, 0))]
        out_shape += [jax.ShapeDtypeStruct((N, D), jnp.float32),
                      jax.ShapeDtypeStruct((N, D), jnp.float32)]
    body = functools.partial(_bnres_body, emit_tables=emit_tables)
    if not emit_tables:
        def body(m_ref, stats_ref, xin_ref, hf_ref, hb_ref, g_ref, be_ref, xout_ref):
            _bnres_body(m_ref, stats_ref, xin_ref, hf_ref, hb_ref, g_ref, be_ref,
                        xout_ref, None, None, emit_tables=False)
    res = pl.pallas_call(
        body,
        grid=(_NBLK,),
        in_specs=[
            pl.BlockSpec((_BLK, D), lambda i: (i, 0)),
            pl.BlockSpec((8, D), lambda i: (0, 0)),
            pl.BlockSpec((_BLK, D), lambda i: (i, 0)),
            pl.BlockSpec((_BLK, D), lambda i: (i, 0)),
            pl.BlockSpec((_BLK, D), lambda i: (i, 0)),
            pl.BlockSpec((1, D), lambda i: (0, 0)),
            pl.BlockSpec((1, D), lambda i: (0, 0)),
        ],
        out_specs=out_specs,
        out_shape=out_shape,
    )(m, stats, xin, hist_all[0:N], hist_all[N:2 * N], g2, be2)
    return res


def kernel(x, fwd_edges_index, bwd_edges_index, edge_attr,
           fwd_W0, fwd_b0, bwd_W0, bwd_b0, mW0, mb0, g0, be0,
           fwd_W1, fwd_b1, bwd_W1, bwd_b1, mW1, mb1, g1, be1):
    src_all = jnp.concatenate([fwd_edges_index[0], bwd_edges_index[0] + N])
    dst_all = jnp.concatenate([fwd_edges_index[1], bwd_edges_index[1]])
    zeros128 = jnp.zeros((NPAD, D), jnp.float32)
    ones128 = jnp.ones((CHUNK, D), jnp.float32)

    hist_pad = _sc_hist(dst_all, zeros128, ones128)
    hist_all = jnp.concatenate([hist_pad[0:N], hist_pad[NPAD:NPAD + N]])

    tcat = _tc_prep(x, hist_all)

    agg_all = _sc_agg(tcat, src_all, dst_all, zeros128)
    m, stats = _tc_merge(agg_all, tcat, hist_all, fwd_W0, fwd_b0, bwd_W0, bwd_b0, mW0, mb0)
    x1, tf1, tb1 = _tc_bnres(m, stats, x, hist_all, g0, be0, True)
    tcat1 = jnp.concatenate([tf1, tb1], axis=0)

    agg_all1 = _sc_agg(tcat1, src_all, dst_all, zeros128)
    m1, stats1 = _tc_merge(agg_all1, tcat1, hist_all, fwd_W1, fwd_b1, bwd_W1, bwd_b1, mW1, mb1)
    (x2,) = _tc_bnres(m1, stats1, x1, hist_all, g1, be1, False)
    return x2


# speedup vs baseline: 21.2364x; 1.3989x over previous
import dataclasses
import functools

import jax
import jax.numpy as jnp
from jax import lax
from jax.experimental import pallas as pl
from jax.experimental.pallas import tpu as pltpu
from jax.experimental.pallas import tpu_sc as plsc

N = 10000
E = 320000
D = 128
NSC = 2
NSUB = 16
EPC = E // NSUB
CHUNK = 200
NCHUNK = EPC // CHUNK
NPAD = 10240
RPS = NPAD // NSUB

@functools.cache
def _get_mesh():
    return plsc.VectorSubcoreMesh(core_axis_name="c", subcore_axis_name="s",
                                  num_cores=NSC, num_subcores=NSUB)


def _sc_compiler_params():
    cp = pltpu.CompilerParams()
    if "needs_layout_passes" in pltpu.CompilerParams.__dataclass_fields__:
        cp = dataclasses.replace(cp, needs_layout_passes=False)
    return cp


def _hist_body(dst_hbm, zeros_hbm, ones_hbm, out_hbm, acc, dstv, ones, sem):
    cid = lax.axis_index("c")
    sid = lax.axis_index("s")

    pltpu.sync_copy(ones_hbm, ones)
    pltpu.sync_copy(zeros_hbm.at[pl.ds(sid * RPS, RPS)], acc.at[pl.ds(sid * RPS, RPS)])
    plsc.subcore_barrier()

    @pl.loop(0, NCHUNK)
    def _(j):
        base = cid * E + sid * EPC + j * CHUNK
        pltpu.sync_copy(dst_hbm.at[pl.ds(base, CHUNK)], dstv)
        pltpu.sync_copy(ones, acc.at[dstv], add=True)

    plsc.subcore_barrier()
    pltpu.sync_copy(acc.at[pl.ds(sid * RPS, RPS)],
                    out_hbm.at[pl.ds(cid * NPAD + sid * RPS, RPS)])


GCH = 80
HALF = EPC // 2
NGC = HALF // GCH
PAIRS = (NGC - 1) // 2


def _agg_body(tcat_hbm, src_hbm, dst3d_hbm, zeros_hbm, out_hbm,
              acc, srcv, dstv, rows0, rows1, sem0, sem1):
    cid = lax.axis_index("c")
    sid = lax.axis_index("s")

    pltpu.sync_copy(zeros_hbm.at[pl.ds(sid * RPS, RPS)], acc.at[pl.ds(sid * RPS, RPS)])
    plsc.subcore_barrier()

    def gather(k, rbuf, sem):
        pltpu.async_copy(tcat_hbm.at[srcv.at[pl.ds(k * GCH, GCH)]], rbuf, sem)

    def gwait(rbuf, sem):
        pltpu.make_async_copy(tcat_hbm.at[srcv.at[pl.ds(0, GCH)]], rbuf, sem).wait()

    for h in range(2):
        base = cid * E + sid * EPC + h * HALF
        pltpu.sync_copy(src_hbm.at[pl.ds(base, HALF)], srcv)
        pltpu.sync_copy(dst3d_hbm.at[2 * (cid * NSUB + sid) + h], dstv)
        gather(0, rows0, sem0)

        @pl.loop(0, PAIRS)
        def _(j):
            k = 2 * j
            gather(k + 1, rows1, sem1)
            gwait(rows0, sem0)
            pltpu.sync_copy(rows0, acc.at[dstv.at[k]], add=True)
            gather(k + 2, rows0, sem0)
            gwait(rows1, sem1)
            pltpu.sync_copy(rows1, acc.at[dstv.at[k + 1]], add=True)

        gwait(rows0, sem0)
        pltpu.sync_copy(rows0, acc.at[dstv.at[NGC - 1]], add=True)

    plsc.subcore_barrier()
    pltpu.sync_copy(acc.at[pl.ds(sid * RPS, RPS)],
                    out_hbm.at[pl.ds(cid * NPAD + sid * RPS, RPS)])


@jax.jit
def _sc_hist(dst_all, zeros128, ones128):
    k = pl.kernel(
        _hist_body,
        out_type=jax.ShapeDtypeStruct((2 * NPAD, D), jnp.float32),
        mesh=_get_mesh(),
        scratch_types=[
            pltpu.VMEM_SHARED((NPAD, D), jnp.float32),
            pltpu.VMEM((CHUNK,), jnp.int32),
            pltpu.VMEM((CHUNK, D), jnp.float32),
            pltpu.SemaphoreType.DMA,
        ],
        compiler_params=_sc_compiler_params(),
    )
    return k(dst_all, zeros128, ones128)


@jax.jit
def _sc_agg(tcat, src_all, dst3d, zeros128):
    k = pl.kernel(
        _agg_body,
        out_type=jax.ShapeDtypeStruct((2 * NPAD, D), jnp.float32),
        mesh=_get_mesh(),
        scratch_types=[
            pltpu.VMEM_SHARED((NPAD, D), jnp.float32),
            pltpu.VMEM((HALF,), jnp.int32),
            pltpu.VMEM((NGC, GCH), jnp.int32),
            pltpu.VMEM((GCH, D), jnp.float32),
            pltpu.VMEM((GCH, D), jnp.float32),
            pltpu.SemaphoreType.DMA,
            pltpu.SemaphoreType.DMA,
        ],
        compiler_params=_sc_compiler_params(),
    )
    return k(tcat, src_all, dst3d, zeros128)


_BLK = 1000
_NBLK = N // _BLK


def _prep_body(x_ref, hist_ref, out_ref):
    dinv = lax.rsqrt(hist_ref[:, 0:1] + 1.0)
    out_ref[...] = dinv * x_ref[...]


@jax.jit
def _tc_prep(x, hist_all):
    return pl.pallas_call(
        _prep_body,
        grid=(2 * _NBLK,),
        in_specs=[
            pl.BlockSpec((_BLK, D), lambda i: (lax.rem(i, _NBLK), 0)),
            pl.BlockSpec((_BLK, D), lambda i: (i, 0)),
        ],
        out_specs=pl.BlockSpec((_BLK, D), lambda i: (i, 0)),
        out_shape=jax.ShapeDtypeStruct((2 * N, D), jnp.float32),
    )(x, hist_all)


def _merge_body(aggf_ref, aggb_ref, tf_ref, tb_ref, hf_ref, hb_ref,
                fW_ref, fb_ref, bW_ref, bb_ref, mW_ref, mb_ref,
                m_ref, stats_ref, acc_ref):
    i = pl.program_id(0)
    dinvf = lax.rsqrt(hf_ref[:, 0:1] + 1.0)
    dinvb = lax.rsqrt(hb_ref[:, 0:1] + 1.0)
    convf = dinvf * jnp.dot(aggf_ref[...] + tf_ref[...], fW_ref[...],
                            preferred_element_type=jnp.float32) + fb_ref[...]
    convb = dinvb * jnp.dot(aggb_ref[...] + tb_ref[...], bW_ref[...],
                            preferred_element_type=jnp.float32) + bb_ref[...]
    m = (jnp.dot(convf, mW_ref[0:D, :], preferred_element_type=jnp.float32)
         + jnp.dot(convb, mW_ref[D:2 * D, :], preferred_element_type=jnp.float32)
         + mb_ref[...])
    m = jnp.maximum(m, 0.0)
    m_ref[...] = m

    @pl.when(i == 0)
    def _():
        acc_ref[...] = jnp.zeros_like(acc_ref)

    s = jnp.sum(m, axis=0, keepdims=True)
    s2 = jnp.sum(m * m, axis=0, keepdims=True)
    acc_ref[0:1, :] += s
    acc_ref[1:2, :] += s2
    stats_ref[...] = acc_ref[...]


@jax.jit
def _tc_merge(agg_all, tcat, hist_all, fW, fb, bW, bb, mW, mb):
    fb2 = fb.reshape(1, D)
    bb2 = bb.reshape(1, D)
    mb2 = mb.reshape(1, D)
    return pl.pallas_call(
        _merge_body,
        grid=(_NBLK,),
        in_specs=[
            pl.BlockSpec((_BLK, D), lambda i: (i, 0)),
            pl.BlockSpec((_BLK, D), lambda i: (i, 0)),
            pl.BlockSpec((_BLK, D), lambda i: (i, 0)),
            pl.BlockSpec((_BLK, D), lambda i: (i, 0)),
            pl.BlockSpec((_BLK, D), lambda i: (i, 0)),
            pl.BlockSpec((_BLK, D), lambda i: (i, 0)),
            pl.BlockSpec((D, D), lambda i: (0, 0)),
            pl.BlockSpec((1, D), lambda i: (0, 0)),
            pl.BlockSpec((D, D), lambda i: (0, 0)),
            pl.BlockSpec((1, D), lambda i: (0, 0)),
            pl.BlockSpec((2 * D, D), lambda i: (0, 0)),
            pl.BlockSpec((1, D), lambda i: (0, 0)),
        ],
        out_specs=[
            pl.BlockSpec((_BLK, D), lambda i: (i, 0)),
            pl.BlockSpec((8, D), lambda i: (0, 0)),
        ],
        out_shape=[
            jax.ShapeDtypeStruct((N, D), jnp.float32),
            jax.ShapeDtypeStruct((8, D), jnp.float32),
        ],
        scratch_shapes=[pltpu.VMEM((8, D), jnp.float32)],
    )(agg_all[0:N], agg_all[NPAD:NPAD + N], tcat[0:N], tcat[N:2 * N],
      hist_all[0:N], hist_all[N:2 * N], fW, fb2, bW, bb2, mW, mb2)


def _bnres_body(m_ref, stats_ref, xin_ref, hf_ref, hb_ref, g_ref, be_ref,
                xout_ref, tf_ref, tb_ref, *, emit_tables):
    mu = stats_ref[0:1, :] * (1.0 / N)
    var = stats_ref[1:2, :] * (1.0 / N) - mu * mu
    y = (m_ref[...] - mu) * lax.rsqrt(var + 1e-5) * g_ref[...] + be_ref[...]
    xout = y + xin_ref[...]
    xout_ref[...] = xout
    if emit_tables:
        tf_ref[...] = lax.rsqrt(hf_ref[:, 0:1] + 1.0) * xout
        tb_ref[...] = lax.rsqrt(hb_ref[:, 0:1] + 1.0) * xout


@functools.partial(jax.jit, static_argnames=("emit_tables",))
def _tc_bnres(m, stats, xin, hist_all, g, be, emit_tables):
    g2 = g.reshape(1, D)
    be2 = be.reshape(1, D)
    out_specs = [pl.BlockSpec((_BLK, D), lambda i: (i, 0))]
    out_shape = [jax.ShapeDtypeStruct((N, D), jnp.float32)]
    if emit_tables:
        out_specs += [pl.BlockSpec((_BLK, D), lambda i: (i, 0)),
                      pl.BlockSpec((_BLK, D), lambda i: (i, 0))]
        out_shape += [jax.ShapeDtypeStruct((N, D), jnp.float32),
                      jax.ShapeDtypeStruct((N, D), jnp.float32)]
    body = functools.partial(_bnres_body, emit_tables=emit_tables)
    if not emit_tables:
        def body(m_ref, stats_ref, xin_ref, hf_ref, hb_ref, g_ref, be_ref, xout_ref):
            _bnres_body(m_ref, stats_ref, xin_ref, hf_ref, hb_ref, g_ref, be_ref,
                        xout_ref, None, None, emit_tables=False)
    res = pl.pallas_call(
        body,
        grid=(_NBLK,),
        in_specs=[
            pl.BlockSpec((_BLK, D), lambda i: (i, 0)),
            pl.BlockSpec((8, D), lambda i: (0, 0)),
            pl.BlockSpec((_BLK, D), lambda i: (i, 0)),
            pl.BlockSpec((_BLK, D), lambda i: (i, 0)),
            pl.BlockSpec((_BLK, D), lambda i: (i, 0)),
            pl.BlockSpec((1, D), lambda i: (0, 0)),
            pl.BlockSpec((1, D), lambda i: (0, 0)),
        ],
        out_specs=out_specs,
        out_shape=out_shape,
    )(m, stats, xin, hist_all[0:N], hist_all[N:2 * N], g2, be2)
    return res


def kernel(x, fwd_edges_index, bwd_edges_index, edge_attr,
           fwd_W0, fwd_b0, bwd_W0, bwd_b0, mW0, mb0, g0, be0,
           fwd_W1, fwd_b1, bwd_W1, bwd_b1, mW1, mb1, g1, be1):
    src_all = jnp.concatenate([fwd_edges_index[0], bwd_edges_index[0] + N])
    dst_all = jnp.concatenate([fwd_edges_index[1], bwd_edges_index[1]])
    zeros128 = jnp.zeros((NPAD, D), jnp.float32)
    ones128 = jnp.ones((CHUNK, D), jnp.float32)

    hist_pad = _sc_hist(dst_all, zeros128, ones128)
    hist_all = jnp.concatenate([hist_pad[0:N], hist_pad[NPAD:NPAD + N]])

    tcat = _tc_prep(x, hist_all)

    dst3d = dst_all.reshape(2 * NSUB * 2, HALF // GCH, GCH)
    agg_all = _sc_agg(tcat, src_all, dst3d, zeros128)
    m, stats = _tc_merge(agg_all, tcat, hist_all, fwd_W0, fwd_b0, bwd_W0, bwd_b0, mW0, mb0)
    x1, tf1, tb1 = _tc_bnres(m, stats, x, hist_all, g0, be0, True)
    tcat1 = jnp.concatenate([tf1, tb1], axis=0)

    agg_all1 = _sc_agg(tcat1, src_all, dst3d, zeros128)
    m1, stats1 = _tc_merge(agg_all1, tcat1, hist_all, fwd_W1, fwd_b1, bwd_W1, bwd_b1, mW1, mb1)
    (x2,) = _tc_bnres(m1, stats1, x1, hist_all, g1, be1, False)
    return x2
